# baseline (speedup 1.0000x reference)
import jax
import jax.numpy as jnp
from jax.experimental import pallas as pl

_TAU0 = 0.5


def _body(logits_ref, u_ref, out_ref):
    l = logits_ref[0]
    u = u_ref[0]
    e2l = jnp.exp(l * (1.0 / _TAU0))
    t = jnp.log(u)
    w = e2l / (t * t)
    s = jnp.sum(w, axis=-1, keepdims=True)
    out_ref[0] = jnp.max(w * (1.0 / s), axis=0, keepdims=True)


def kernel(logits, uniform):
    B, D = logits.shape
    _, K, _ = uniform.shape
    out = pl.pallas_call(
        _body,
        grid=(B,),
        in_specs=[
            pl.BlockSpec((1, 1, D), lambda b: (b, 0, 0)),
            pl.BlockSpec((1, K, D), lambda b: (b, 0, 0)),
        ],
        out_specs=pl.BlockSpec((1, 1, D), lambda b: (b, 0, 0)),
        out_shape=jax.ShapeDtypeStruct((B, 1, D), jnp.float32),
    )(logits.reshape(B, 1, D), uniform)
    return out.reshape(B, D)


# speedup vs baseline: 1.0429x; 1.0429x over previous
import jax
import jax.numpy as jnp
from jax.experimental import pallas as pl
from jax.experimental.pallas import tpu as pltpu

_TAU0 = 0.5
_BB = 2


def _body(logits_ref, u_ref, out_ref):
    for i in range(_BB):
        l = logits_ref[i]
        u = u_ref[i]
        e2l = jnp.exp(l * (1.0 / _TAU0))
        t = jnp.log(u)
        w = e2l / (t * t)
        s = jnp.sum(w, axis=-1, keepdims=True)
        out_ref[i] = jnp.max(w * (1.0 / s), axis=0, keepdims=True)


def kernel(logits, uniform):
    B, D = logits.shape
    _, K, _ = uniform.shape
    out = pl.pallas_call(
        _body,
        grid=(B // _BB,),
        in_specs=[
            pl.BlockSpec((_BB, 1, D), lambda b: (b, 0, 0)),
            pl.BlockSpec((_BB, K, D), lambda b: (b, 0, 0)),
        ],
        out_specs=pl.BlockSpec((_BB, 1, D), lambda b: (b, 0, 0)),
        out_shape=jax.ShapeDtypeStruct((B, 1, D), jnp.float32),
        compiler_params=pltpu.CompilerParams(
            dimension_semantics=("arbitrary",),
            vmem_limit_bytes=100 * 1024 * 1024,
        ),
    )(logits.reshape(B, 1, D), uniform)
    return out.reshape(B, D)
